# baseline (speedup 1.0000x reference)
import jax
import jax.numpy as jnp
from jax.experimental import pallas as pl
from jax.experimental.pallas import tpu as pltpu

_BLOCK = 1000


def _fused_body(mb0_ref, mb1_ref, h_ref, e_ref, p_ref, wh_ref, wm_ref, we_ref, wc_ref, b_ref, out_ref):
    msg = jnp.sum(mb0_ref[...], axis=1) + jnp.sum(mb1_ref[...], axis=1)
    p = p_ref[...]
    ids = jax.lax.broadcasted_iota(jnp.int32, p.shape, 1)
    maxv = jnp.max(p, axis=1, keepdims=True)
    cls = jnp.min(jnp.where(p == maxv, ids, p.shape[1]), axis=1).astype(jnp.float32)
    acc = jnp.dot(h_ref[...], wh_ref[...], preferred_element_type=jnp.float32)
    acc = acc + jnp.dot(msg, wm_ref[...], preferred_element_type=jnp.float32)
    acc = acc + jnp.dot(e_ref[...], we_ref[...], preferred_element_type=jnp.float32)
    acc = acc + cls[:, None] * wc_ref[...] + b_ref[...]
    out_ref[...] = jnp.maximum(acc, 0.0)


def kernel(mailbox_edge_message, node_hidden_rep, node_type_embedding, node_prediction, W, b):
    N, K, d_msg = mailbox_edge_message.shape
    d_h = node_hidden_rep.shape[1]
    d_e = node_type_embedding.shape[1]
    d_out = W.shape[0]
    Wt = W.T
    wh = Wt[:d_h]
    wm = Wt[d_h:d_h + d_msg]
    we = Wt[d_h + d_msg:d_h + d_msg + d_e]
    wc = Wt[d_h + d_msg + d_e:]
    b2 = b[None, :]

    grid = (N // _BLOCK,)
    return pl.pallas_call(
        _fused_body,
        grid=grid,
        in_specs=[
            pl.BlockSpec((_BLOCK, K // 2, d_msg), lambda i: (i, 0, 0)),
            pl.BlockSpec((_BLOCK, K // 2, d_msg), lambda i: (i, 1, 0)),
            pl.BlockSpec((_BLOCK, d_h), lambda i: (i, 0)),
            pl.BlockSpec((_BLOCK, d_e), lambda i: (i, 0)),
            pl.BlockSpec((_BLOCK, node_prediction.shape[1]), lambda i: (i, 0)),
            pl.BlockSpec((d_h, d_out), lambda i: (0, 0)),
            pl.BlockSpec((d_msg, d_out), lambda i: (0, 0)),
            pl.BlockSpec((d_e, d_out), lambda i: (0, 0)),
            pl.BlockSpec((1, d_out), lambda i: (0, 0)),
            pl.BlockSpec((1, d_out), lambda i: (0, 0)),
        ],
        out_specs=pl.BlockSpec((_BLOCK, d_out), lambda i: (i, 0)),
        out_shape=jax.ShapeDtypeStruct((N, d_out), jnp.float32),
        compiler_params=pltpu.CompilerParams(
            dimension_semantics=("arbitrary",),
        ),
    )(mailbox_edge_message, mailbox_edge_message, node_hidden_rep,
      node_type_embedding, node_prediction, wh, wm, we, wc, b2)


# speedup vs baseline: 1.0958x; 1.0958x over previous
import jax
import jax.numpy as jnp
from jax.experimental import pallas as pl
from jax.experimental.pallas import tpu as pltpu

_BLOCK = 1000


def _fused_body(mb_ref, h_ref, e_ref, p_ref, wh_ref, wm_ref, we_ref, wc_ref, b_ref, out_ref):
    msg = jnp.sum(mb_ref[...], axis=1)
    p = p_ref[...]
    ids = jax.lax.broadcasted_iota(jnp.int32, p.shape, 1)
    maxv = jnp.max(p, axis=1, keepdims=True)
    cls = jnp.min(jnp.where(p == maxv, ids, p.shape[1]), axis=1).astype(jnp.float32)
    acc = jnp.dot(h_ref[...].astype(jnp.bfloat16), wh_ref[...],
                  preferred_element_type=jnp.float32)
    acc = acc + jnp.dot(msg.astype(jnp.bfloat16), wm_ref[...],
                        preferred_element_type=jnp.float32)
    acc = acc + jnp.dot(e_ref[...].astype(jnp.bfloat16), we_ref[...],
                        preferred_element_type=jnp.float32)
    acc = acc + cls[:, None] * wc_ref[...] + b_ref[...]
    out_ref[...] = jnp.maximum(acc, 0.0)


def kernel(mailbox_edge_message, node_hidden_rep, node_type_embedding, node_prediction, W, b):
    N, K, d_msg = mailbox_edge_message.shape
    d_h = node_hidden_rep.shape[1]
    d_e = node_type_embedding.shape[1]
    d_out = W.shape[0]
    Wt = W.T
    wh = Wt[:d_h].astype(jnp.bfloat16)
    wm = Wt[d_h:d_h + d_msg].astype(jnp.bfloat16)
    we = Wt[d_h + d_msg:d_h + d_msg + d_e].astype(jnp.bfloat16)
    wc = Wt[d_h + d_msg + d_e:]
    b2 = b[None, :]

    grid = (N // _BLOCK,)
    return pl.pallas_call(
        _fused_body,
        grid=grid,
        in_specs=[
            pl.BlockSpec((_BLOCK, K, d_msg), lambda i: (i, 0, 0)),
            pl.BlockSpec((_BLOCK, d_h), lambda i: (i, 0)),
            pl.BlockSpec((_BLOCK, d_e), lambda i: (i, 0)),
            pl.BlockSpec((_BLOCK, node_prediction.shape[1]), lambda i: (i, 0)),
            pl.BlockSpec((d_h, d_out), lambda i: (0, 0)),
            pl.BlockSpec((d_msg, d_out), lambda i: (0, 0)),
            pl.BlockSpec((d_e, d_out), lambda i: (0, 0)),
            pl.BlockSpec((1, d_out), lambda i: (0, 0)),
            pl.BlockSpec((1, d_out), lambda i: (0, 0)),
        ],
        out_specs=pl.BlockSpec((_BLOCK, d_out), lambda i: (i, 0)),
        out_shape=jax.ShapeDtypeStruct((N, d_out), jnp.float32),
        compiler_params=pltpu.CompilerParams(
            dimension_semantics=("arbitrary",),
        ),
    )(mailbox_edge_message, node_hidden_rep, node_type_embedding,
      node_prediction, wh, wm, we, wc, b2)
